# Initial kernel scaffold; baseline (speedup 1.0000x reference)
import jax
import jax.numpy as jnp
from jax.experimental import pallas as pl


def kernel(cond_ids, cat_ids, cond_table, cat_table, cat_start):
    raise NotImplementedError("write your pallas kernel here")


# speedup vs baseline: 6.3930x; 6.3930x over previous
import functools

import jax
import jax.numpy as jnp
from jax import lax
from jax.experimental import pallas as pl
from jax.experimental.pallas import tpu as pltpu
from jax.experimental.pallas import tpu_sc as plsc

D = 128
L = 16
NC = 2
NS = 16
NW = NC * NS
CHUNK = 128


def _sc_embed(cond_flat, cat_flat, cond_table, cat_table, cat_start_pad):
    B = cond_flat.shape[0]
    b_per_w = B // NW
    n_chunks = b_per_w // CHUNK
    n_cs = cat_start_pad.shape[0]
    mesh = plsc.VectorSubcoreMesh(core_axis_name="c", subcore_axis_name="s")

    @functools.partial(
        pl.kernel,
        out_type=jax.ShapeDtypeStruct((B, D), jnp.float32),
        mesh=mesh,
        compiler_params=pltpu.CompilerParams(needs_layout_passes=False),
        scratch_types=[
            pltpu.VMEM((n_cs,), jnp.int32),
            pltpu.VMEM((CHUNK,), jnp.int32),
            pltpu.VMEM((CHUNK,), jnp.int32),
            pltpu.VMEM((CHUNK,), jnp.int32),
            pltpu.VMEM((CHUNK, D), jnp.float32),
            pltpu.VMEM((CHUNK, D), jnp.float32),
            pltpu.SemaphoreType.DMA,
            pltpu.SemaphoreType.DMA,
        ],
    )
    def k(cond_hbm, cat_hbm, condtab_hbm, cattab_hbm, cs_hbm, out_hbm,
          cs_v, cond_v, cat_v, full_v, rows_cond, rows_cat, sem0, sem1):
        wid = lax.axis_index("s") * NC + lax.axis_index("c")
        base = wid * b_per_w
        pltpu.sync_copy(cs_hbm, cs_v)

        def chunk_body(g, carry):
            off = base + g * CHUNK
            pltpu.sync_copy(cond_hbm.at[pl.ds(off, CHUNK)], cond_v)
            pltpu.sync_copy(cat_hbm.at[pl.ds(off, CHUNK)], cat_v)
            for j in range(CHUNK // L):
                sl = pl.ds(j * L, L)
                starts = plsc.load_gather(cs_v, [cond_v[sl]])
                full_v[sl] = starts + cat_v[sl]
            cp_cat = pltpu.async_copy(cattab_hbm.at[full_v], rows_cat, sem0)
            cp_cond = pltpu.async_copy(condtab_hbm.at[cond_v], rows_cond, sem1)
            cp_cat.wait()
            cp_cond.wait()

            def add_body(r, c):
                for kk in range(D // L):
                    sl = pl.ds(kk * L, L)
                    rows_cat[r, sl] = rows_cat[r, sl] + rows_cond[r, sl]
                return c

            lax.fori_loop(0, CHUNK, add_body, 0)
            pltpu.sync_copy(rows_cat, out_hbm.at[pl.ds(off, CHUNK)])
            return carry

        lax.fori_loop(0, n_chunks, chunk_body, 0)

    return k(cond_flat, cat_flat, cond_table, cat_table, cat_start_pad)


def kernel(cond_ids, cat_ids, cond_table, cat_table, cat_start):
    bt, f = cond_ids.shape
    cond_flat = cond_ids.reshape(-1).astype(jnp.int32)
    cat_flat = cat_ids.reshape(-1).astype(jnp.int32)
    cs = cat_start.astype(jnp.int32)
    n_pad = ((cs.shape[0] + 7) // 8) * 8
    cs_pad = jnp.zeros((n_pad,), jnp.int32).at[: cs.shape[0]].set(cs)
    out = _sc_embed(cond_flat, cat_flat, cond_table, cat_table, cs_pad)
    return out.reshape(bt, f, cond_table.shape[1])
